# baseline (speedup 1.0000x reference)
import functools

import jax
import jax.numpy as jnp
from jax import lax
from jax.experimental import pallas as pl
from jax.experimental.pallas import tpu as pltpu
from jax.experimental.pallas import tpu_sc as plsc

N_NODES = 10000
N_EDGES = 320000
D_IN = 128
D_HID = 256
D_OUT = 12
N_GRAPHS = 64

NC = 2
NS = 16
LANES = 16

EROWS = 2560
E_PAD = EROWS * 128
RPT = EROWS // NS
NACC = 10112
ZROWS = NACC // NS


IBL = 32
NCHUNK = RPT // IBL


def _agg_body(t0, t1, src2, dst2, z2, sums_out, sidx, didx, rb, acc, gsem):
    cid = lax.axis_index("c")
    sid = lax.axis_index("s")

    pltpu.sync_copy(z2.at[pl.ds(sid * ZROWS, ZROWS)],
                    acc.at[pl.ds(sid * ZROWS, ZROWS)])
    plsc.subcore_barrier()

    def chunk(c, _):
        base = sid * RPT + c * IBL
        pltpu.sync_copy(src2.at[pl.ds(base, IBL)], sidx)
        pltpu.sync_copy(dst2.at[pl.ds(base, IBL)], didx)

        def step(j, _):
            @pl.when(cid == 0)
            def _():
                pltpu.async_copy(t0.at[sidx.at[j]], rb, gsem).wait()

            @pl.when(cid == 1)
            def _():
                pltpu.async_copy(t1.at[sidx.at[j]], rb, gsem).wait()

            pltpu.sync_copy(rb, acc.at[didx.at[j]], add=True)
            return 0

        lax.fori_loop(0, IBL, step, 0)
        return 0

    lax.fori_loop(0, NCHUNK, chunk, 0)
    plsc.subcore_barrier()

    pltpu.sync_copy(acc.at[pl.ds(sid * ZROWS, ZROWS)],
                    sums_out.at[cid, pl.ds(sid * ZROWS, ZROWS)])


HRPT = EROWS // (NC * NS)


def _agg_counts_body(t, src2, dst2, z2, z1, sums_out, cnt_out,
                     sidx, didx, rb, onesv, acc, accc, gsem):
    cid = lax.axis_index("c")
    sid = lax.axis_index("s")

    pltpu.sync_copy(z2.at[pl.ds(sid * ZROWS, ZROWS)],
                    acc.at[pl.ds(sid * ZROWS, ZROWS)])

    @pl.when(sid == 0)
    def _():
        pltpu.sync_copy(z1, accc)

    for i in range(128 // LANES):
        onesv[pl.ds(i * LANES, LANES)] = jnp.ones((LANES,), jnp.float32)

    base = (cid * NS + sid) * HRPT
    pltpu.sync_copy(src2.at[pl.ds(base, HRPT)], sidx)
    pltpu.sync_copy(dst2.at[pl.ds(base, HRPT)], didx)
    plsc.subcore_barrier()

    def step(j, _):
        pltpu.async_copy(t.at[sidx.at[j]], rb, gsem).wait()
        pltpu.sync_copy(rb, acc.at[didx.at[j]], add=True)
        pltpu.sync_copy(onesv, accc.at[didx.at[j]], add=True)
        return 0

    lax.fori_loop(0, HRPT, step, 0)
    plsc.subcore_barrier()

    pltpu.sync_copy(acc.at[pl.ds(sid * ZROWS, ZROWS)],
                    sums_out.at[cid, pl.ds(sid * ZROWS, ZROWS)])

    @pl.when(sid == 0)
    def _():
        pltpu.sync_copy(accc, cnt_out.at[pl.ds(cid * NACC, NACC)])


def _make_agg(C, with_counts):
    mesh = plsc.VectorSubcoreMesh(core_axis_name="c", subcore_axis_name="s")
    if with_counts:
        out_type = (jax.ShapeDtypeStruct((NC, NACC, C), jnp.float32),
                    jax.ShapeDtypeStruct((NC * NACC,), jnp.float32))
        scratch = [
            pltpu.VMEM((HRPT, 128), jnp.int32),
            pltpu.VMEM((HRPT, 128), jnp.int32),
            pltpu.VMEM((128, C), jnp.float32),
            pltpu.VMEM((128,), jnp.float32),
            pltpu.VMEM_SHARED((NACC, C), jnp.float32),
            pltpu.VMEM_SHARED((NACC,), jnp.float32),
            pltpu.SemaphoreType.DMA,
        ]
        return pl.kernel(_agg_counts_body, out_type=out_type, mesh=mesh,
                         scratch_types=scratch)
    out_type = jax.ShapeDtypeStruct((NC, NACC, C), jnp.float32)
    scratch = [
        pltpu.VMEM((IBL, 128), jnp.int32),
        pltpu.VMEM((IBL, 128), jnp.int32),
        pltpu.VMEM((128, C), jnp.float32),
        pltpu.VMEM_SHARED((NACC, C), jnp.float32),
        pltpu.SemaphoreType.DMA,
    ]
    return pl.kernel(_agg_body, out_type=out_type, mesh=mesh,
                     scratch_types=scratch)


ROWS_TC = 1000
GRID_TC = N_NODES // ROWS_TC


def _dense1_body(s0, s1, c0, c1, x, wl, wr, b, oa, ob):
    inv = 1.0 / jnp.maximum(c0[...] + c1[...], 1.0)
    mean = (s0[...] + s1[...]) * inv
    h = jnp.dot(mean, wl[...], preferred_element_type=jnp.float32)
    h += jnp.dot(x[...], wr[...], preferred_element_type=jnp.float32)
    h = jnp.maximum(h + b[...], 0.0)
    oa[...] = h[:, :D_IN]
    ob[...] = h[:, D_IN:]


def _dense2_body(s0, s1, c0, c1, h1a, h1b, bat, wl, wr, b, wlin, blin,
                 out, pooled):
    i = pl.program_id(0)

    @pl.when(i == 0)
    def _():
        pooled[...] = jnp.zeros_like(pooled)

    inv = 1.0 / jnp.maximum(c0[...] + c1[...], 1.0)
    mean = jnp.concatenate([s0[...], s1[...]], axis=1) * inv
    h1 = jnp.concatenate([h1a[...], h1b[...]], axis=1)
    h = jnp.dot(mean, wl[...], preferred_element_type=jnp.float32)
    h += jnp.dot(h1, wr[...], preferred_element_type=jnp.float32)
    h = jnp.maximum(h + b[...], 0.0)
    oh = jnp.equal(
        bat[...],
        lax.broadcasted_iota(jnp.int32, (ROWS_TC, N_GRAPHS), 1),
    ).astype(jnp.float32)
    pooled[...] += lax.dot_general(oh, h, (((0,), (0,)), ((), ())),
                                   preferred_element_type=jnp.float32)

    @pl.when(i == GRID_TC - 1)
    def _():
        out[...] = (jnp.dot(pooled[...], wlin[...],
                            preferred_element_type=jnp.float32) + blin[...])


def _row_spec(cols):
    return pl.BlockSpec((ROWS_TC, cols), lambda i: (i, 0))


def _full_spec(r, c):
    return pl.BlockSpec((r, c), lambda i: (0, 0))


_dense1 = pl.pallas_call(
    _dense1_body,
    grid=(GRID_TC,),
    in_specs=[
        _row_spec(D_IN), _row_spec(D_IN), _row_spec(1), _row_spec(1),
        _row_spec(D_IN),
        _full_spec(D_IN, D_HID), _full_spec(D_IN, D_HID), _full_spec(1, D_HID),
    ],
    out_specs=[_row_spec(D_IN), _row_spec(D_IN)],
    out_shape=[jax.ShapeDtypeStruct((N_NODES, D_IN), jnp.float32),
               jax.ShapeDtypeStruct((N_NODES, D_IN), jnp.float32)],
)

_dense2 = pl.pallas_call(
    _dense2_body,
    grid=(GRID_TC,),
    in_specs=[
        _row_spec(D_IN), _row_spec(D_IN), _row_spec(1), _row_spec(1),
        _row_spec(D_IN), _row_spec(D_IN), _row_spec(1),
        _full_spec(D_HID, D_HID), _full_spec(D_HID, D_HID),
        _full_spec(1, D_HID), _full_spec(D_HID, 128), _full_spec(1, 128),
    ],
    out_specs=_full_spec(N_GRAPHS, 128),
    out_shape=jax.ShapeDtypeStruct((N_GRAPHS, 128), jnp.float32),
    scratch_shapes=[pltpu.VMEM((N_GRAPHS, D_HID), jnp.float32)],
)

_agg1 = _make_agg(128, with_counts=True)
_agg128 = _make_agg(128, with_counts=False)


@jax.jit
def kernel(x, edge_index, batch, W1l, b1, W1r, W2l, b2, W2r, Wlin, blin):
    src = edge_index[0]
    dst = edge_index[1]
    pad = E_PAD - N_EDGES
    src2 = jnp.concatenate([src, jnp.zeros((pad,), jnp.int32)]).reshape(EROWS, 128)
    pad_dst = N_NODES + (jnp.arange(pad, dtype=jnp.int32) % (NACC - N_NODES))
    dst2 = jnp.concatenate([dst, pad_dst]).reshape(EROWS, 128)

    z128 = jnp.zeros((NACC, 128), jnp.float32)
    z1 = jnp.zeros((NACC,), jnp.float32)

    sums1, cnt = _agg1(x, src2, dst2, z128, z1)
    cnt = cnt.reshape(NC, NACC)
    c0 = cnt[0].reshape(NACC, 1)
    c1 = cnt[1].reshape(NACC, 1)

    h1a, h1b = _dense1(sums1[0], sums1[1], c0, c1, x, W1l, W1r,
                       b1.reshape(1, D_HID))

    sums2 = _agg128(h1a, h1b, src2, dst2, z128)

    outp = _dense2(sums2[0], sums2[1], c0, c1, h1a, h1b,
                   batch.reshape(N_NODES, 1).astype(jnp.int32),
                   W2l, W2r, b2.reshape(1, D_HID),
                   jnp.pad(Wlin, ((0, 0), (0, 128 - D_OUT))),
                   jnp.pad(blin, (0, 128 - D_OUT)).reshape(1, 128))
    return outp[:, :D_OUT]


# speedup vs baseline: 1.0959x; 1.0959x over previous
import functools

import jax
import jax.numpy as jnp
from jax import lax
from jax.experimental import pallas as pl
from jax.experimental.pallas import tpu as pltpu
from jax.experimental.pallas import tpu_sc as plsc

N_NODES = 10000
N_EDGES = 320000
D_IN = 128
D_HID = 256
D_OUT = 12
N_GRAPHS = 64

NC = 2
NS = 16
LANES = 16

EROWS = 2560
E_PAD = EROWS * 128
RPT = EROWS // NS
NACC = 10112
ZROWS = NACC // NS


IBC = 8
HRPT = EROWS // (NC * NS)


def _edge_loop(gather_fn, nrow, src2, dst2, idx_base, acc, accc_ones,
               sidxs, didxs, rbs, gsem, ssems):
    npair = nrow // (2 * IBC)

    def pair(p, _):
        for half in range(2):
            c = 2 * p + half
            base = idx_base + c * IBC
            pltpu.sync_copy(src2.at[pl.ds(base, IBC)], sidxs[half])
            pltpu.sync_copy(dst2.at[pl.ds(base, IBC)], didxs[half])
            for i in range(IBC):
                b = i % 2
                ob = 1 - b
                gather_fn(sidxs[half].at[i], rbs[b], gsem)
                oh = half if i > 0 else 1 - half
                oi = (i - 1) % IBC
                def wait_prev():
                    pltpu.make_async_copy(
                        rbs[ob], acc.at[didxs[oh].at[oi]], ssems[ob]).wait()
                    if accc_ones is not None:
                        accc, onesv = accc_ones
                        pltpu.make_async_copy(
                            onesv, accc.at[didxs[oh].at[oi]], ssems[ob]).wait()
                if half == 0 and i == 0:
                    @pl.when(p > 0)
                    def _():
                        wait_prev()
                else:
                    wait_prev()
                pltpu.async_copy(rbs[b], acc.at[didxs[half].at[i]],
                                 ssems[b], add=True)
                if accc_ones is not None:
                    accc, onesv = accc_ones
                    pltpu.async_copy(onesv, accc.at[didxs[half].at[i]],
                                     ssems[b], add=True)
        return 0

    lax.fori_loop(0, npair, pair, 0)
    lb = (IBC - 1) % 2
    pltpu.make_async_copy(rbs[lb], acc.at[didxs[1].at[IBC - 1]],
                          ssems[lb]).wait()
    if accc_ones is not None:
        accc, onesv = accc_ones
        pltpu.make_async_copy(onesv, accc.at[didxs[1].at[IBC - 1]],
                              ssems[lb]).wait()


def _agg_body(t0, t1, src2, dst2, z2, sums_out,
              sidxA, sidxB, didxA, didxB, rbA, rbB,
              acc, gsem, ssemA, ssemB):
    cid = lax.axis_index("c")
    sid = lax.axis_index("s")

    pltpu.sync_copy(z2.at[pl.ds(sid * ZROWS, ZROWS)],
                    acc.at[pl.ds(sid * ZROWS, ZROWS)])
    plsc.subcore_barrier()

    def gather_fn(idx_row, rb, sem):
        @pl.when(cid == 0)
        def _():
            pltpu.async_copy(t0.at[idx_row], rb, sem).wait()

        @pl.when(cid == 1)
        def _():
            pltpu.async_copy(t1.at[idx_row], rb, sem).wait()

    _edge_loop(gather_fn, RPT, src2, dst2, sid * RPT, acc, None,
               (sidxA, sidxB), (didxA, didxB), (rbA, rbB),
               gsem, (ssemA, ssemB))
    plsc.subcore_barrier()

    pltpu.sync_copy(acc.at[pl.ds(sid * ZROWS, ZROWS)],
                    sums_out.at[cid, pl.ds(sid * ZROWS, ZROWS)])


def _agg_counts_body(t, src2, dst2, z2, z1, sums_out, cnt_out,
                     sidxA, sidxB, didxA, didxB, rbA, rbB, onesv,
                     acc, accc, gsem, ssemA, ssemB):
    cid = lax.axis_index("c")
    sid = lax.axis_index("s")

    pltpu.sync_copy(z2.at[pl.ds(sid * ZROWS, ZROWS)],
                    acc.at[pl.ds(sid * ZROWS, ZROWS)])

    @pl.when(sid == 0)
    def _():
        pltpu.sync_copy(z1, accc)

    for i in range(128 // LANES):
        onesv[pl.ds(i * LANES, LANES)] = jnp.ones((LANES,), jnp.float32)

    plsc.subcore_barrier()

    def gather_fn(idx_row, rb, sem):
        pltpu.async_copy(t.at[idx_row], rb, sem).wait()

    _edge_loop(gather_fn, HRPT, src2, dst2, (cid * NS + sid) * HRPT,
               acc, (accc, onesv),
               (sidxA, sidxB), (didxA, didxB), (rbA, rbB),
               gsem, (ssemA, ssemB))
    plsc.subcore_barrier()

    pltpu.sync_copy(acc.at[pl.ds(sid * ZROWS, ZROWS)],
                    sums_out.at[cid, pl.ds(sid * ZROWS, ZROWS)])

    @pl.when(sid == 0)
    def _():
        pltpu.sync_copy(accc, cnt_out.at[pl.ds(cid * NACC, NACC)])


def _make_agg(C, with_counts):
    mesh = plsc.VectorSubcoreMesh(core_axis_name="c", subcore_axis_name="s")
    idx = [pltpu.VMEM((IBC, 128), jnp.int32)] * 4
    rbs = [pltpu.VMEM((128, C), jnp.float32)] * 2
    sems = [pltpu.SemaphoreType.DMA] * 3
    if with_counts:
        out_type = (jax.ShapeDtypeStruct((NC, NACC, C), jnp.float32),
                    jax.ShapeDtypeStruct((NC * NACC,), jnp.float32))
        scratch = idx + rbs + [
            pltpu.VMEM((128,), jnp.float32),
            pltpu.VMEM_SHARED((NACC, C), jnp.float32),
            pltpu.VMEM_SHARED((NACC,), jnp.float32),
        ] + sems
        return pl.kernel(_agg_counts_body, out_type=out_type, mesh=mesh,
                         scratch_types=scratch)
    out_type = jax.ShapeDtypeStruct((NC, NACC, C), jnp.float32)
    scratch = idx + rbs + [
        pltpu.VMEM_SHARED((NACC, C), jnp.float32),
    ] + sems
    return pl.kernel(_agg_body, out_type=out_type, mesh=mesh,
                     scratch_types=scratch)


ROWS_TC = 1000
GRID_TC = N_NODES // ROWS_TC


def _dense1_body(s0, s1, c0, c1, x, wl, wr, b, oa, ob):
    inv = 1.0 / jnp.maximum(c0[...] + c1[...], 1.0)
    mean = (s0[...] + s1[...]) * inv
    h = jnp.dot(mean, wl[...], preferred_element_type=jnp.float32)
    h += jnp.dot(x[...], wr[...], preferred_element_type=jnp.float32)
    h = jnp.maximum(h + b[...], 0.0)
    oa[...] = h[:, :D_IN]
    ob[...] = h[:, D_IN:]


def _dense2_body(s0, s1, c0, c1, h1a, h1b, bat, wl, wr, b, wlin, blin,
                 out, pooled):
    i = pl.program_id(0)

    @pl.when(i == 0)
    def _():
        pooled[...] = jnp.zeros_like(pooled)

    inv = 1.0 / jnp.maximum(c0[...] + c1[...], 1.0)
    mean = jnp.concatenate([s0[...], s1[...]], axis=1) * inv
    h1 = jnp.concatenate([h1a[...], h1b[...]], axis=1)
    h = jnp.dot(mean, wl[...], preferred_element_type=jnp.float32)
    h += jnp.dot(h1, wr[...], preferred_element_type=jnp.float32)
    h = jnp.maximum(h + b[...], 0.0)
    oh = jnp.equal(
        bat[...],
        lax.broadcasted_iota(jnp.int32, (ROWS_TC, N_GRAPHS), 1),
    ).astype(jnp.float32)
    pooled[...] += lax.dot_general(oh, h, (((0,), (0,)), ((), ())),
                                   preferred_element_type=jnp.float32)

    @pl.when(i == GRID_TC - 1)
    def _():
        out[...] = (jnp.dot(pooled[...], wlin[...],
                            preferred_element_type=jnp.float32) + blin[...])


def _row_spec(cols):
    return pl.BlockSpec((ROWS_TC, cols), lambda i: (i, 0))


def _full_spec(r, c):
    return pl.BlockSpec((r, c), lambda i: (0, 0))


_dense1 = pl.pallas_call(
    _dense1_body,
    grid=(GRID_TC,),
    in_specs=[
        _row_spec(D_IN), _row_spec(D_IN), _row_spec(1), _row_spec(1),
        _row_spec(D_IN),
        _full_spec(D_IN, D_HID), _full_spec(D_IN, D_HID), _full_spec(1, D_HID),
    ],
    out_specs=[_row_spec(D_IN), _row_spec(D_IN)],
    out_shape=[jax.ShapeDtypeStruct((N_NODES, D_IN), jnp.float32),
               jax.ShapeDtypeStruct((N_NODES, D_IN), jnp.float32)],
)

_dense2 = pl.pallas_call(
    _dense2_body,
    grid=(GRID_TC,),
    in_specs=[
        _row_spec(D_IN), _row_spec(D_IN), _row_spec(1), _row_spec(1),
        _row_spec(D_IN), _row_spec(D_IN), _row_spec(1),
        _full_spec(D_HID, D_HID), _full_spec(D_HID, D_HID),
        _full_spec(1, D_HID), _full_spec(D_HID, 128), _full_spec(1, 128),
    ],
    out_specs=_full_spec(N_GRAPHS, 128),
    out_shape=jax.ShapeDtypeStruct((N_GRAPHS, 128), jnp.float32),
    scratch_shapes=[pltpu.VMEM((N_GRAPHS, D_HID), jnp.float32)],
)

_agg1 = _make_agg(128, with_counts=True)
_agg128 = _make_agg(128, with_counts=False)


@jax.jit
def kernel(x, edge_index, batch, W1l, b1, W1r, W2l, b2, W2r, Wlin, blin):
    src = edge_index[0]
    dst = edge_index[1]
    pad = E_PAD - N_EDGES
    src2 = jnp.concatenate([src, jnp.zeros((pad,), jnp.int32)]).reshape(EROWS, 128)
    pad_dst = N_NODES + (jnp.arange(pad, dtype=jnp.int32) % (NACC - N_NODES))
    dst2 = jnp.concatenate([dst, pad_dst]).reshape(EROWS, 128)

    z128 = jnp.zeros((NACC, 128), jnp.float32)
    z1 = jnp.zeros((NACC,), jnp.float32)

    sums1, cnt = _agg1(x, src2, dst2, z128, z1)
    cnt = cnt.reshape(NC, NACC)
    c0 = cnt[0].reshape(NACC, 1)
    c1 = cnt[1].reshape(NACC, 1)

    h1a, h1b = _dense1(sums1[0], sums1[1], c0, c1, x, W1l, W1r,
                       b1.reshape(1, D_HID))

    sums2 = _agg128(h1a, h1b, src2, dst2, z128)

    outp = _dense2(sums2[0], sums2[1], c0, c1, h1a, h1b,
                   batch.reshape(N_NODES, 1).astype(jnp.int32),
                   W2l, W2r, b2.reshape(1, D_HID),
                   jnp.pad(Wlin, ((0, 0), (0, 128 - D_OUT))),
                   jnp.pad(blin, (0, 128 - D_OUT)).reshape(1, 128))
    return outp[:, :D_OUT]


# speedup vs baseline: 1.0963x; 1.0003x over previous
import functools

import jax
import jax.numpy as jnp
from jax import lax
from jax.experimental import pallas as pl
from jax.experimental.pallas import tpu as pltpu
from jax.experimental.pallas import tpu_sc as plsc

N_NODES = 10000
N_EDGES = 320000
D_IN = 128
D_HID = 256
D_OUT = 12
N_GRAPHS = 64

NC = 2
NS = 16
LANES = 16

EROWS = 2560
E_PAD = EROWS * 128
RPT = EROWS // NS
NACC = 10112
ZROWS = NACC // NS


IBC = 8
HRPT = EROWS // (NC * NS)


def _edge_loop(gfire, gwait, nrow, src2, dst2, idx_base, acc, accc_ones,
               sidxs, didxs, rbs, gsem, ssems):
    npair = nrow // (2 * IBC)

    pltpu.sync_copy(src2.at[pl.ds(idx_base, IBC)], sidxs[0])
    pltpu.sync_copy(dst2.at[pl.ds(idx_base, IBC)], didxs[0])
    gfire(sidxs[0].at[0], rbs[0], gsem)

    def pair(p, _):
        for half in range(2):
            c = 2 * p + half
            for i in range(IBC):
                b = i % 2
                ob = 1 - b
                oh = half if i > 0 else 1 - half
                oi = (i - 1) % IBC

                gwait(rbs[b], gsem)

                def wait_prev():
                    pltpu.make_async_copy(
                        rbs[ob], acc.at[didxs[oh].at[oi]], ssems[ob]).wait()
                    if accc_ones is not None:
                        accc, onesv = accc_ones
                        pltpu.make_async_copy(
                            onesv, accc.at[didxs[oh].at[oi]], ssems[ob]).wait()

                if half == 0 and i == 0:
                    @pl.when(p > 0)
                    def _():
                        wait_prev()
                else:
                    wait_prev()

                if i < IBC - 1:
                    gfire(sidxs[half].at[i + 1], rbs[ob], gsem)
                elif half == 0:
                    base = idx_base + (c + 1) * IBC
                    pltpu.sync_copy(src2.at[pl.ds(base, IBC)], sidxs[1])
                    pltpu.sync_copy(dst2.at[pl.ds(base, IBC)], didxs[1])
                    gfire(sidxs[1].at[0], rbs[ob], gsem)
                else:
                    @pl.when(p < npair - 1)
                    def _():
                        base = idx_base + (c + 1) * IBC
                        pltpu.sync_copy(src2.at[pl.ds(base, IBC)], sidxs[0])
                        pltpu.sync_copy(dst2.at[pl.ds(base, IBC)], didxs[0])
                        gfire(sidxs[0].at[0], rbs[ob], gsem)

                pltpu.async_copy(rbs[b], acc.at[didxs[half].at[i]],
                                 ssems[b], add=True)
                if accc_ones is not None:
                    accc, onesv = accc_ones
                    pltpu.async_copy(onesv, accc.at[didxs[half].at[i]],
                                     ssems[b], add=True)
        return 0

    lax.fori_loop(0, npair, pair, 0)
    lb = (IBC - 1) % 2
    pltpu.make_async_copy(rbs[lb], acc.at[didxs[1].at[IBC - 1]],
                          ssems[lb]).wait()
    if accc_ones is not None:
        accc, onesv = accc_ones
        pltpu.make_async_copy(onesv, accc.at[didxs[1].at[IBC - 1]],
                              ssems[lb]).wait()


def _agg_body(t0, t1, src2, dst2, z2, sums_out,
              sidxA, sidxB, didxA, didxB, rbA, rbB,
              acc, gsem, ssemA, ssemB):
    cid = lax.axis_index("c")
    sid = lax.axis_index("s")

    pltpu.sync_copy(z2.at[pl.ds(sid * ZROWS, ZROWS)],
                    acc.at[pl.ds(sid * ZROWS, ZROWS)])
    plsc.subcore_barrier()

    def gfire(idx_row, rb, sem):
        @pl.when(cid == 0)
        def _():
            pltpu.async_copy(t0.at[idx_row], rb, sem)

        @pl.when(cid == 1)
        def _():
            pltpu.async_copy(t1.at[idx_row], rb, sem)

    def gwait(rb, sem):
        pltpu.make_async_copy(t0.at[sidxA.at[0]], rb, sem).wait()

    _edge_loop(gfire, gwait, RPT, src2, dst2, sid * RPT, acc, None,
               (sidxA, sidxB), (didxA, didxB), (rbA, rbB),
               gsem, (ssemA, ssemB))
    plsc.subcore_barrier()

    pltpu.sync_copy(acc.at[pl.ds(sid * ZROWS, ZROWS)],
                    sums_out.at[cid, pl.ds(sid * ZROWS, ZROWS)])


def _agg_counts_body(t, src2, dst2, z2, z1, sums_out, cnt_out,
                     sidxA, sidxB, didxA, didxB, rbA, rbB, onesv,
                     acc, accc, gsem, ssemA, ssemB):
    cid = lax.axis_index("c")
    sid = lax.axis_index("s")

    pltpu.sync_copy(z2.at[pl.ds(sid * ZROWS, ZROWS)],
                    acc.at[pl.ds(sid * ZROWS, ZROWS)])

    @pl.when(sid == 0)
    def _():
        pltpu.sync_copy(z1, accc)

    for i in range(128 // LANES):
        onesv[pl.ds(i * LANES, LANES)] = jnp.ones((LANES,), jnp.float32)

    plsc.subcore_barrier()

    def gfire(idx_row, rb, sem):
        pltpu.async_copy(t.at[idx_row], rb, sem)

    def gwait(rb, sem):
        pltpu.make_async_copy(t.at[sidxA.at[0]], rb, sem).wait()

    _edge_loop(gfire, gwait, HRPT, src2, dst2, (cid * NS + sid) * HRPT,
               acc, (accc, onesv),
               (sidxA, sidxB), (didxA, didxB), (rbA, rbB),
               gsem, (ssemA, ssemB))
    plsc.subcore_barrier()

    pltpu.sync_copy(acc.at[pl.ds(sid * ZROWS, ZROWS)],
                    sums_out.at[cid, pl.ds(sid * ZROWS, ZROWS)])

    @pl.when(sid == 0)
    def _():
        pltpu.sync_copy(accc, cnt_out.at[pl.ds(cid * NACC, NACC)])


def _make_agg(C, with_counts):
    mesh = plsc.VectorSubcoreMesh(core_axis_name="c", subcore_axis_name="s")
    idx = [pltpu.VMEM((IBC, 128), jnp.int32)] * 4
    rbs = [pltpu.VMEM((128, C), jnp.float32)] * 2
    sems = [pltpu.SemaphoreType.DMA] * 3
    if with_counts:
        out_type = (jax.ShapeDtypeStruct((NC, NACC, C), jnp.float32),
                    jax.ShapeDtypeStruct((NC * NACC,), jnp.float32))
        scratch = idx + rbs + [
            pltpu.VMEM((128,), jnp.float32),
            pltpu.VMEM_SHARED((NACC, C), jnp.float32),
            pltpu.VMEM_SHARED((NACC,), jnp.float32),
        ] + sems
        return pl.kernel(_agg_counts_body, out_type=out_type, mesh=mesh,
                         scratch_types=scratch)
    out_type = jax.ShapeDtypeStruct((NC, NACC, C), jnp.float32)
    scratch = idx + rbs + [
        pltpu.VMEM_SHARED((NACC, C), jnp.float32),
    ] + sems
    return pl.kernel(_agg_body, out_type=out_type, mesh=mesh,
                     scratch_types=scratch)


ROWS_TC = 1000
GRID_TC = N_NODES // ROWS_TC


def _dense1_body(s0, s1, c0, c1, x, wl, wr, b, oa, ob):
    inv = 1.0 / jnp.maximum(c0[...] + c1[...], 1.0)
    mean = (s0[...] + s1[...]) * inv
    h = jnp.dot(mean, wl[...], preferred_element_type=jnp.float32)
    h += jnp.dot(x[...], wr[...], preferred_element_type=jnp.float32)
    h = jnp.maximum(h + b[...], 0.0)
    oa[...] = h[:, :D_IN]
    ob[...] = h[:, D_IN:]


def _dense2_body(s0, s1, c0, c1, h1a, h1b, bat, wl, wr, b, wlin, blin,
                 out, pooled):
    i = pl.program_id(0)

    @pl.when(i == 0)
    def _():
        pooled[...] = jnp.zeros_like(pooled)

    inv = 1.0 / jnp.maximum(c0[...] + c1[...], 1.0)
    mean = jnp.concatenate([s0[...], s1[...]], axis=1) * inv
    h1 = jnp.concatenate([h1a[...], h1b[...]], axis=1)
    h = jnp.dot(mean, wl[...], preferred_element_type=jnp.float32)
    h += jnp.dot(h1, wr[...], preferred_element_type=jnp.float32)
    h = jnp.maximum(h + b[...], 0.0)
    oh = jnp.equal(
        bat[...],
        lax.broadcasted_iota(jnp.int32, (ROWS_TC, N_GRAPHS), 1),
    ).astype(jnp.float32)
    pooled[...] += lax.dot_general(oh, h, (((0,), (0,)), ((), ())),
                                   preferred_element_type=jnp.float32)

    @pl.when(i == GRID_TC - 1)
    def _():
        out[...] = (jnp.dot(pooled[...], wlin[...],
                            preferred_element_type=jnp.float32) + blin[...])


def _row_spec(cols):
    return pl.BlockSpec((ROWS_TC, cols), lambda i: (i, 0))


def _full_spec(r, c):
    return pl.BlockSpec((r, c), lambda i: (0, 0))


_dense1 = pl.pallas_call(
    _dense1_body,
    grid=(GRID_TC,),
    in_specs=[
        _row_spec(D_IN), _row_spec(D_IN), _row_spec(1), _row_spec(1),
        _row_spec(D_IN),
        _full_spec(D_IN, D_HID), _full_spec(D_IN, D_HID), _full_spec(1, D_HID),
    ],
    out_specs=[_row_spec(D_IN), _row_spec(D_IN)],
    out_shape=[jax.ShapeDtypeStruct((N_NODES, D_IN), jnp.float32),
               jax.ShapeDtypeStruct((N_NODES, D_IN), jnp.float32)],
)

_dense2 = pl.pallas_call(
    _dense2_body,
    grid=(GRID_TC,),
    in_specs=[
        _row_spec(D_IN), _row_spec(D_IN), _row_spec(1), _row_spec(1),
        _row_spec(D_IN), _row_spec(D_IN), _row_spec(1),
        _full_spec(D_HID, D_HID), _full_spec(D_HID, D_HID),
        _full_spec(1, D_HID), _full_spec(D_HID, 128), _full_spec(1, 128),
    ],
    out_specs=_full_spec(N_GRAPHS, 128),
    out_shape=jax.ShapeDtypeStruct((N_GRAPHS, 128), jnp.float32),
    scratch_shapes=[pltpu.VMEM((N_GRAPHS, D_HID), jnp.float32)],
)

_agg1 = _make_agg(128, with_counts=True)
_agg128 = _make_agg(128, with_counts=False)


@jax.jit
def kernel(x, edge_index, batch, W1l, b1, W1r, W2l, b2, W2r, Wlin, blin):
    src = edge_index[0]
    dst = edge_index[1]
    pad = E_PAD - N_EDGES
    src2 = jnp.concatenate([src, jnp.zeros((pad,), jnp.int32)]).reshape(EROWS, 128)
    pad_dst = N_NODES + (jnp.arange(pad, dtype=jnp.int32) % (NACC - N_NODES))
    dst2 = jnp.concatenate([dst, pad_dst]).reshape(EROWS, 128)

    z128 = jnp.zeros((NACC, 128), jnp.float32)
    z1 = jnp.zeros((NACC,), jnp.float32)

    sums1, cnt = _agg1(x, src2, dst2, z128, z1)
    cnt = cnt.reshape(NC, NACC)
    c0 = cnt[0].reshape(NACC, 1)
    c1 = cnt[1].reshape(NACC, 1)

    h1a, h1b = _dense1(sums1[0], sums1[1], c0, c1, x, W1l, W1r,
                       b1.reshape(1, D_HID))

    sums2 = _agg128(h1a, h1b, src2, dst2, z128)

    outp = _dense2(sums2[0], sums2[1], c0, c1, h1a, h1b,
                   batch.reshape(N_NODES, 1).astype(jnp.int32),
                   W2l, W2r, b2.reshape(1, D_HID),
                   jnp.pad(Wlin, ((0, 0), (0, 128 - D_OUT))),
                   jnp.pad(blin, (0, 128 - D_OUT)).reshape(1, 128))
    return outp[:, :D_OUT]


# speedup vs baseline: 2.6271x; 2.3964x over previous
import functools

import jax
import jax.numpy as jnp
from jax import lax
from jax.experimental import pallas as pl
from jax.experimental.pallas import tpu as pltpu
from jax.experimental.pallas import tpu_sc as plsc

N_NODES = 10000
N_EDGES = 320000
D_IN = 128
D_HID = 256
D_OUT = 12
N_GRAPHS = 64

NC = 2
NS = 16
LANES = 16

EROWS = 2560
E_PAD = EROWS * 128
RPT = EROWS // NS
NACC = 10112
ZROWS = NACC // NS


IBC = 8
HRPT = EROWS // (NC * NS)


def _edge_loop(gfire, gwait, nrow, src2, dst2, idx_base, acc, accc_ones,
               sidxs, didxs, rbs, gsem, ssems):
    npair = nrow // (2 * IBC)

    pltpu.sync_copy(src2.at[pl.ds(idx_base, IBC)], sidxs[0])
    pltpu.sync_copy(dst2.at[pl.ds(idx_base, IBC)], didxs[0])
    gfire(sidxs[0].at[0], rbs[0], gsem)

    def pair(p, _):
        for half in range(2):
            c = 2 * p + half
            for i in range(IBC):
                b = i % 2
                ob = 1 - b
                oh = half if i > 0 else 1 - half
                oi = (i - 1) % IBC

                gwait(rbs[b], gsem)

                def wait_prev():
                    pltpu.make_async_copy(
                        rbs[ob], acc.at[didxs[oh].at[oi]], ssems[ob]).wait()
                    if accc_ones is not None:
                        accc, onesv = accc_ones
                        pltpu.make_async_copy(
                            onesv, accc.at[didxs[oh].at[oi]], ssems[ob]).wait()

                if half == 0 and i == 0:
                    @pl.when(p > 0)
                    def _():
                        wait_prev()
                else:
                    wait_prev()

                if i < IBC - 1:
                    gfire(sidxs[half].at[i + 1], rbs[ob], gsem)
                elif half == 0:
                    base = idx_base + (c + 1) * IBC
                    pltpu.sync_copy(src2.at[pl.ds(base, IBC)], sidxs[1])
                    pltpu.sync_copy(dst2.at[pl.ds(base, IBC)], didxs[1])
                    gfire(sidxs[1].at[0], rbs[ob], gsem)
                else:
                    @pl.when(p < npair - 1)
                    def _():
                        base = idx_base + (c + 1) * IBC
                        pltpu.sync_copy(src2.at[pl.ds(base, IBC)], sidxs[0])
                        pltpu.sync_copy(dst2.at[pl.ds(base, IBC)], didxs[0])
                        gfire(sidxs[0].at[0], rbs[ob], gsem)

                pltpu.async_copy(rbs[b], acc.at[didxs[half].at[i]],
                                 ssems[b], add=True)
                if accc_ones is not None:
                    accc, onesv = accc_ones
                    pltpu.async_copy(onesv, accc.at[didxs[half].at[i]],
                                     ssems[b], add=True)
        return 0

    lax.fori_loop(0, npair, pair, 0)
    lb = (IBC - 1) % 2
    pltpu.make_async_copy(rbs[lb], acc.at[didxs[1].at[IBC - 1]],
                          ssems[lb]).wait()
    if accc_ones is not None:
        accc, onesv = accc_ones
        pltpu.make_async_copy(onesv, accc.at[didxs[1].at[IBC - 1]],
                              ssems[lb]).wait()


def _agg_body(t0, t1, src2, dst2, z2, sums_out,
              sidxA, sidxB, didxA, didxB, rbA, rbB,
              acc, gsem, ssemA, ssemB):
    cid = lax.axis_index("c")
    sid = lax.axis_index("s")

    pltpu.sync_copy(z2.at[pl.ds(sid * ZROWS, ZROWS)],
                    acc.at[pl.ds(sid * ZROWS, ZROWS)])
    plsc.subcore_barrier()

    def gfire(idx_row, rb, sem):
        @pl.when(cid == 0)
        def _():
            pltpu.async_copy(t0.at[idx_row], rb, sem)

        @pl.when(cid == 1)
        def _():
            pltpu.async_copy(t1.at[idx_row], rb, sem)

    def gwait(rb, sem):
        pltpu.make_async_copy(t0.at[sidxA.at[0]], rb, sem).wait()

    _edge_loop(gfire, gwait, RPT, src2, dst2, sid * RPT, acc, None,
               (sidxA, sidxB), (didxA, didxB), (rbA, rbB),
               gsem, (ssemA, ssemB))
    plsc.subcore_barrier()

    pltpu.sync_copy(acc.at[pl.ds(sid * ZROWS, ZROWS)],
                    sums_out.at[cid, pl.ds(sid * ZROWS, ZROWS)])


def _agg_counts_body(t, src2, dst2, z2, z1, sums_out, cnt_out,
                     sidxA, sidxB, didxA, didxB, rbA, rbB, onesv,
                     acc, accc, gsem, ssemA, ssemB):
    cid = lax.axis_index("c")
    sid = lax.axis_index("s")

    pltpu.sync_copy(z2.at[pl.ds(sid * ZROWS, ZROWS)],
                    acc.at[pl.ds(sid * ZROWS, ZROWS)])

    @pl.when(sid == 0)
    def _():
        pltpu.sync_copy(z1, accc)

    for i in range(128 // LANES):
        onesv[pl.ds(i * LANES, LANES)] = jnp.ones((LANES,), jnp.float32)

    plsc.subcore_barrier()

    def gfire(idx_row, rb, sem):
        pltpu.async_copy(t.at[idx_row], rb, sem)

    def gwait(rb, sem):
        pltpu.make_async_copy(t.at[sidxA.at[0]], rb, sem).wait()

    _edge_loop(gfire, gwait, HRPT, src2, dst2, (cid * NS + sid) * HRPT,
               acc, (accc, onesv),
               (sidxA, sidxB), (didxA, didxB), (rbA, rbB),
               gsem, (ssemA, ssemB))
    plsc.subcore_barrier()

    pltpu.sync_copy(acc.at[pl.ds(sid * ZROWS, ZROWS)],
                    sums_out.at[cid, pl.ds(sid * ZROWS, ZROWS)])

    @pl.when(sid == 0)
    def _():
        pltpu.sync_copy(accc, cnt_out.at[pl.ds(cid * NACC, NACC)])


def _make_agg(C, with_counts):
    mesh = plsc.VectorSubcoreMesh(core_axis_name="c", subcore_axis_name="s")
    idx = [pltpu.VMEM((IBC, 128), jnp.int32)] * 4
    rbs = [pltpu.VMEM((128, C), jnp.float32)] * 2
    sems = [pltpu.SemaphoreType.DMA] * 3
    if with_counts:
        out_type = (jax.ShapeDtypeStruct((NC, NACC, C), jnp.float32),
                    jax.ShapeDtypeStruct((NC * NACC,), jnp.float32))
        scratch = idx + rbs + [
            pltpu.VMEM((128,), jnp.float32),
            pltpu.VMEM_SHARED((NACC, C), jnp.float32),
            pltpu.VMEM_SHARED((NACC,), jnp.float32),
        ] + sems
        return pl.kernel(_agg_counts_body, out_type=out_type, mesh=mesh,
                         scratch_types=scratch)
    out_type = jax.ShapeDtypeStruct((NC, NACC, C), jnp.float32)
    scratch = idx + rbs + [
        pltpu.VMEM_SHARED((NACC, C), jnp.float32),
    ] + sems
    return pl.kernel(_agg_body, out_type=out_type, mesh=mesh,
                     scratch_types=scratch)


ROWS_TC = 1000
GRID_TC = N_NODES // ROWS_TC


def _dense1_body(s0, s1, c0, c1, x, wl, wr, b, oa, ob):
    inv = 1.0 / jnp.maximum(c0[...] + c1[...], 1.0)
    mean = (s0[...] + s1[...]) * inv
    h = jnp.dot(mean, wl[...], preferred_element_type=jnp.float32)
    h += jnp.dot(x[...], wr[...], preferred_element_type=jnp.float32)
    h = jnp.maximum(h + b[...], 0.0)
    oa[...] = h[:, :D_IN]
    ob[...] = h[:, D_IN:]


def _dense2_body(s0, s1, c0, c1, h1a, h1b, bat, wl, wr, b, wlin, blin,
                 out, pooled):
    i = pl.program_id(0)

    @pl.when(i == 0)
    def _():
        pooled[...] = jnp.zeros_like(pooled)

    inv = 1.0 / jnp.maximum(c0[...] + c1[...], 1.0)
    mean = jnp.concatenate([s0[...], s1[...]], axis=1) * inv
    h1 = jnp.concatenate([h1a[...], h1b[...]], axis=1)
    h = jnp.dot(mean, wl[...], preferred_element_type=jnp.float32)
    h += jnp.dot(h1, wr[...], preferred_element_type=jnp.float32)
    h = jnp.maximum(h + b[...], 0.0)
    oh = jnp.equal(
        bat[...],
        lax.broadcasted_iota(jnp.int32, (ROWS_TC, N_GRAPHS), 1),
    ).astype(jnp.float32)
    pooled[...] += lax.dot_general(oh, h, (((0,), (0,)), ((), ())),
                                   preferred_element_type=jnp.float32)

    @pl.when(i == GRID_TC - 1)
    def _():
        out[...] = (jnp.dot(pooled[...], wlin[...],
                            preferred_element_type=jnp.float32) + blin[...])


def _row_spec(cols):
    return pl.BlockSpec((ROWS_TC, cols), lambda i: (i, 0))


def _full_spec(r, c):
    return pl.BlockSpec((r, c), lambda i: (0, 0))


_dense1 = pl.pallas_call(
    _dense1_body,
    grid=(GRID_TC,),
    in_specs=[
        _row_spec(D_IN), _row_spec(D_IN), _row_spec(1), _row_spec(1),
        _row_spec(D_IN),
        _full_spec(D_IN, D_HID), _full_spec(D_IN, D_HID), _full_spec(1, D_HID),
    ],
    out_specs=[_row_spec(D_IN), _row_spec(D_IN)],
    out_shape=[jax.ShapeDtypeStruct((N_NODES, D_IN), jnp.float32),
               jax.ShapeDtypeStruct((N_NODES, D_IN), jnp.float32)],
)

_dense2 = pl.pallas_call(
    _dense2_body,
    grid=(GRID_TC,),
    in_specs=[
        _row_spec(D_IN), _row_spec(D_IN), _row_spec(1), _row_spec(1),
        _row_spec(D_IN), _row_spec(D_IN), _row_spec(1),
        _full_spec(D_HID, D_HID), _full_spec(D_HID, D_HID),
        _full_spec(1, D_HID), _full_spec(D_HID, 128), _full_spec(1, 128),
    ],
    out_specs=_full_spec(N_GRAPHS, 128),
    out_shape=jax.ShapeDtypeStruct((N_GRAPHS, 128), jnp.float32),
    scratch_shapes=[pltpu.VMEM((N_GRAPHS, D_HID), jnp.float32)],
)

_agg1 = _make_agg(128, with_counts=True)
_agg128 = _make_agg(128, with_counts=False)


@jax.jit
def kernel(x, edge_index, batch, W1l, b1, W1r, W2l, b2, W2r, Wlin, blin):
    src = edge_index[0]
    dst = edge_index[1]
    pad = E_PAD - N_EDGES
    pad_src = jnp.arange(pad, dtype=jnp.int32) % N_NODES
    src2 = jnp.concatenate([src, pad_src]).reshape(EROWS, 128)
    pad_dst = N_NODES + (jnp.arange(pad, dtype=jnp.int32) % (NACC - N_NODES))
    dst2 = jnp.concatenate([dst, pad_dst]).reshape(EROWS, 128)

    z128 = jnp.zeros((NACC, 128), jnp.float32)
    z1 = jnp.zeros((NACC,), jnp.float32)

    sums1, cnt = _agg1(x, src2, dst2, z128, z1)
    cnt = cnt.reshape(NC, NACC)
    c0 = cnt[0].reshape(NACC, 1)
    c1 = cnt[1].reshape(NACC, 1)

    h1a, h1b = _dense1(sums1[0], sums1[1], c0, c1, x, W1l, W1r,
                       b1.reshape(1, D_HID))

    sums2 = _agg128(h1a, h1b, src2, dst2, z128)

    outp = _dense2(sums2[0], sums2[1], c0, c1, h1a, h1b,
                   batch.reshape(N_NODES, 1).astype(jnp.int32),
                   W2l, W2r, b2.reshape(1, D_HID),
                   jnp.pad(Wlin, ((0, 0), (0, 128 - D_OUT))),
                   jnp.pad(blin, (0, 128 - D_OUT)).reshape(1, 128))
    return outp[:, :D_OUT]


# speedup vs baseline: 2.7764x; 1.0568x over previous
import functools

import jax
import jax.numpy as jnp
from jax import lax
from jax.experimental import pallas as pl
from jax.experimental.pallas import tpu as pltpu
from jax.experimental.pallas import tpu_sc as plsc

N_NODES = 10000
N_EDGES = 320000
D_IN = 128
D_HID = 256
D_OUT = 12
N_GRAPHS = 64

NC = 2
NS = 16
LANES = 16

EROWS = 5120
E_PAD = EROWS * 64
RPT = EROWS // NS
NACC = 10112
ZROWS = NACC // NS


IBC = 8
HRPT = EROWS // (NC * NS)


def _edge_loop(gfire, gwait, nrow, src2, dst2, idx_base, acc, accc_ones,
               sidxs, didxs, rbs, gsem, ssems):
    npair = nrow // (2 * IBC)

    pltpu.sync_copy(src2.at[pl.ds(idx_base, IBC)], sidxs[0])
    pltpu.sync_copy(dst2.at[pl.ds(idx_base, IBC)], didxs[0])
    gfire(sidxs[0].at[0], rbs[0], gsem)
    gfire(sidxs[0].at[1], rbs[1], gsem)

    def pair(p, _):
        for half in range(2):
            c = 2 * p + half
            for i in range(IBC):
                b = i % 4
                nb = (i + 2) % 4
                oh = half if i > 1 else 1 - half
                oi = (i - 2) % IBC

                gwait(rbs[b], gsem)

                def wait_prev():
                    pltpu.make_async_copy(
                        rbs[nb], acc.at[didxs[oh].at[oi]], ssems[nb]).wait()
                    if accc_ones is not None:
                        accc, onesv = accc_ones
                        pltpu.make_async_copy(
                            onesv, accc.at[didxs[oh].at[oi]], ssems[nb]).wait()

                if half == 0 and i < 2:
                    @pl.when(p > 0)
                    def _():
                        wait_prev()
                else:
                    wait_prev()

                if i < IBC - 2:
                    gfire(sidxs[half].at[i + 2], rbs[nb], gsem)
                elif half == 0:
                    if i == IBC - 2:
                        base = idx_base + (c + 1) * IBC
                        pltpu.sync_copy(src2.at[pl.ds(base, IBC)], sidxs[1])
                        pltpu.sync_copy(dst2.at[pl.ds(base, IBC)], didxs[1])
                    gfire(sidxs[1].at[i - (IBC - 2)], rbs[nb], gsem)
                else:
                    @pl.when(p < npair - 1)
                    def _():
                        if i == IBC - 2:
                            base = idx_base + (c + 1) * IBC
                            pltpu.sync_copy(src2.at[pl.ds(base, IBC)], sidxs[0])
                            pltpu.sync_copy(dst2.at[pl.ds(base, IBC)], didxs[0])
                        gfire(sidxs[0].at[i - (IBC - 2)], rbs[nb], gsem)

                pltpu.async_copy(rbs[b], acc.at[didxs[half].at[i]],
                                 ssems[b], add=True)
                if accc_ones is not None:
                    accc, onesv = accc_ones
                    pltpu.async_copy(onesv, accc.at[didxs[half].at[i]],
                                     ssems[b], add=True)
        return 0

    lax.fori_loop(0, npair, pair, 0)
    for i in (IBC - 2, IBC - 1):
        lb = i % 4
        pltpu.make_async_copy(rbs[lb], acc.at[didxs[1].at[i]],
                              ssems[lb]).wait()
        if accc_ones is not None:
            accc, onesv = accc_ones
            pltpu.make_async_copy(onesv, accc.at[didxs[1].at[i]],
                                  ssems[lb]).wait()


def _agg_body(t0, t1, src2, dst2, z2, sums_out,
              sidxA, sidxB, didxA, didxB, rb0, rb1, rb2, rb3,
              acc, gsem, ssem0, ssem1, ssem2, ssem3):
    cid = lax.axis_index("c")
    sid = lax.axis_index("s")

    pltpu.sync_copy(z2.at[pl.ds(sid * ZROWS, ZROWS)],
                    acc.at[pl.ds(sid * ZROWS, ZROWS)])
    plsc.subcore_barrier()

    def gfire(idx_row, rb, sem):
        @pl.when(cid == 0)
        def _():
            pltpu.async_copy(t0.at[idx_row], rb, sem)

        @pl.when(cid == 1)
        def _():
            pltpu.async_copy(t1.at[idx_row], rb, sem)

    def gwait(rb, sem):
        pltpu.make_async_copy(t0.at[sidxA.at[0]], rb, sem).wait()

    _edge_loop(gfire, gwait, RPT, src2, dst2, sid * RPT, acc, None,
               (sidxA, sidxB), (didxA, didxB), (rb0, rb1, rb2, rb3),
               gsem, (ssem0, ssem1, ssem2, ssem3))
    plsc.subcore_barrier()

    pltpu.sync_copy(acc.at[pl.ds(sid * ZROWS, ZROWS)],
                    sums_out.at[cid, pl.ds(sid * ZROWS, ZROWS)])


def _agg_counts_body(t, src2, dst2, z2, z1, sums_out, cnt_out,
                     sidxA, sidxB, didxA, didxB, rb0, rb1, rb2, rb3, onesv,
                     acc, accc, gsem, ssem0, ssem1, ssem2, ssem3):
    cid = lax.axis_index("c")
    sid = lax.axis_index("s")

    pltpu.sync_copy(z2.at[pl.ds(sid * ZROWS, ZROWS)],
                    acc.at[pl.ds(sid * ZROWS, ZROWS)])

    @pl.when(sid == 0)
    def _():
        pltpu.sync_copy(z1, accc)

    for i in range(64 // LANES):
        onesv[pl.ds(i * LANES, LANES)] = jnp.ones((LANES,), jnp.float32)

    plsc.subcore_barrier()

    def gfire(idx_row, rb, sem):
        pltpu.async_copy(t.at[idx_row], rb, sem)

    def gwait(rb, sem):
        pltpu.make_async_copy(t.at[sidxA.at[0]], rb, sem).wait()

    _edge_loop(gfire, gwait, HRPT, src2, dst2, (cid * NS + sid) * HRPT,
               acc, (accc, onesv),
               (sidxA, sidxB), (didxA, didxB), (rb0, rb1, rb2, rb3),
               gsem, (ssem0, ssem1, ssem2, ssem3))
    plsc.subcore_barrier()

    pltpu.sync_copy(acc.at[pl.ds(sid * ZROWS, ZROWS)],
                    sums_out.at[cid, pl.ds(sid * ZROWS, ZROWS)])

    @pl.when(sid == 0)
    def _():
        pltpu.sync_copy(accc, cnt_out.at[pl.ds(cid * NACC, NACC)])


def _make_agg(C, with_counts):
    mesh = plsc.VectorSubcoreMesh(core_axis_name="c", subcore_axis_name="s")
    idx = [pltpu.VMEM((IBC, 64), jnp.int32)] * 4
    rbs = [pltpu.VMEM((64, C), jnp.float32)] * 4
    sems = [pltpu.SemaphoreType.DMA] * 5
    if with_counts:
        out_type = (jax.ShapeDtypeStruct((NC, NACC, C), jnp.float32),
                    jax.ShapeDtypeStruct((NC * NACC,), jnp.float32))
        scratch = idx + rbs + [
            pltpu.VMEM((64,), jnp.float32),
            pltpu.VMEM_SHARED((NACC, C), jnp.float32),
            pltpu.VMEM_SHARED((NACC,), jnp.float32),
        ] + sems
        return pl.kernel(_agg_counts_body, out_type=out_type, mesh=mesh,
                         scratch_types=scratch)
    out_type = jax.ShapeDtypeStruct((NC, NACC, C), jnp.float32)
    scratch = idx + rbs + [
        pltpu.VMEM_SHARED((NACC, C), jnp.float32),
    ] + sems
    return pl.kernel(_agg_body, out_type=out_type, mesh=mesh,
                     scratch_types=scratch)


ROWS_TC = 1000
GRID_TC = N_NODES // ROWS_TC


def _dense1_body(s0, s1, c0, c1, x, wl, wr, b, oa, ob):
    inv = 1.0 / jnp.maximum(c0[...] + c1[...], 1.0)
    mean = (s0[...] + s1[...]) * inv
    h = jnp.dot(mean, wl[...], preferred_element_type=jnp.float32)
    h += jnp.dot(x[...], wr[...], preferred_element_type=jnp.float32)
    h = jnp.maximum(h + b[...], 0.0)
    oa[...] = h[:, :D_IN]
    ob[...] = h[:, D_IN:]


def _dense2_body(s0, s1, c0, c1, h1a, h1b, bat, wl, wr, b, wlin, blin,
                 out, pooled):
    i = pl.program_id(0)

    @pl.when(i == 0)
    def _():
        pooled[...] = jnp.zeros_like(pooled)

    inv = 1.0 / jnp.maximum(c0[...] + c1[...], 1.0)
    mean = jnp.concatenate([s0[...], s1[...]], axis=1) * inv
    h1 = jnp.concatenate([h1a[...], h1b[...]], axis=1)
    h = jnp.dot(mean, wl[...], preferred_element_type=jnp.float32)
    h += jnp.dot(h1, wr[...], preferred_element_type=jnp.float32)
    h = jnp.maximum(h + b[...], 0.0)
    oh = jnp.equal(
        bat[...],
        lax.broadcasted_iota(jnp.int32, (ROWS_TC, N_GRAPHS), 1),
    ).astype(jnp.float32)
    pooled[...] += lax.dot_general(oh, h, (((0,), (0,)), ((), ())),
                                   preferred_element_type=jnp.float32)

    @pl.when(i == GRID_TC - 1)
    def _():
        out[...] = (jnp.dot(pooled[...], wlin[...],
                            preferred_element_type=jnp.float32) + blin[...])


def _row_spec(cols):
    return pl.BlockSpec((ROWS_TC, cols), lambda i: (i, 0))


def _full_spec(r, c):
    return pl.BlockSpec((r, c), lambda i: (0, 0))


_dense1 = pl.pallas_call(
    _dense1_body,
    grid=(GRID_TC,),
    in_specs=[
        _row_spec(D_IN), _row_spec(D_IN), _row_spec(1), _row_spec(1),
        _row_spec(D_IN),
        _full_spec(D_IN, D_HID), _full_spec(D_IN, D_HID), _full_spec(1, D_HID),
    ],
    out_specs=[_row_spec(D_IN), _row_spec(D_IN)],
    out_shape=[jax.ShapeDtypeStruct((N_NODES, D_IN), jnp.float32),
               jax.ShapeDtypeStruct((N_NODES, D_IN), jnp.float32)],
)

_dense2 = pl.pallas_call(
    _dense2_body,
    grid=(GRID_TC,),
    in_specs=[
        _row_spec(D_IN), _row_spec(D_IN), _row_spec(1), _row_spec(1),
        _row_spec(D_IN), _row_spec(D_IN), _row_spec(1),
        _full_spec(D_HID, D_HID), _full_spec(D_HID, D_HID),
        _full_spec(1, D_HID), _full_spec(D_HID, 128), _full_spec(1, 128),
    ],
    out_specs=_full_spec(N_GRAPHS, 128),
    out_shape=jax.ShapeDtypeStruct((N_GRAPHS, 128), jnp.float32),
    scratch_shapes=[pltpu.VMEM((N_GRAPHS, D_HID), jnp.float32)],
)

_agg1 = _make_agg(128, with_counts=True)
_agg128 = _make_agg(128, with_counts=False)


@jax.jit
def kernel(x, edge_index, batch, W1l, b1, W1r, W2l, b2, W2r, Wlin, blin):
    src = edge_index[0]
    dst = edge_index[1]
    pad = E_PAD - N_EDGES
    pad_src = jnp.arange(pad, dtype=jnp.int32) % N_NODES
    src2 = jnp.concatenate([src, pad_src]).reshape(EROWS, 64)
    pad_dst = N_NODES + (jnp.arange(pad, dtype=jnp.int32) % (NACC - N_NODES))
    dst2 = jnp.concatenate([dst, pad_dst]).reshape(EROWS, 64)

    z128 = jnp.zeros((NACC, 128), jnp.float32)
    z1 = jnp.zeros((NACC,), jnp.float32)

    sums1, cnt = _agg1(x, src2, dst2, z128, z1)
    cnt = cnt.reshape(NC, NACC)
    c0 = cnt[0].reshape(NACC, 1)
    c1 = cnt[1].reshape(NACC, 1)

    h1a, h1b = _dense1(sums1[0], sums1[1], c0, c1, x, W1l, W1r,
                       b1.reshape(1, D_HID))

    sums2 = _agg128(h1a, h1b, src2, dst2, z128)

    outp = _dense2(sums2[0], sums2[1], c0, c1, h1a, h1b,
                   batch.reshape(N_NODES, 1).astype(jnp.int32),
                   W2l, W2r, b2.reshape(1, D_HID),
                   jnp.pad(Wlin, ((0, 0), (0, 128 - D_OUT))),
                   jnp.pad(blin, (0, 128 - D_OUT)).reshape(1, 128))
    return outp[:, :D_OUT]


# speedup vs baseline: 2.8608x; 1.0304x over previous
import functools

import jax
import jax.numpy as jnp
from jax import lax
from jax.experimental import pallas as pl
from jax.experimental.pallas import tpu as pltpu
from jax.experimental.pallas import tpu_sc as plsc

N_NODES = 10000
N_EDGES = 320000
D_IN = 128
D_HID = 256
D_OUT = 12
N_GRAPHS = 64

NC = 2
NS = 16
LANES = 16

EROWS = 5120
E_PAD = EROWS * 64
RPT = EROWS // NS
NACC = 10112
ZROWS = NACC // NS


IBC = 8
HRPT = EROWS // (NC * NS)


def _edge_loop(gfire, gwait, nrow, src2, dst2, idx_base, acc, accc_ones,
               sidxs, didxs, rbs, gsem, ssems):
    npair = nrow // (2 * IBC)

    pltpu.sync_copy(src2.at[pl.ds(idx_base, IBC)], sidxs[0])
    pltpu.sync_copy(dst2.at[pl.ds(idx_base, IBC)], didxs[0])
    gfire(sidxs[0].at[0], rbs[0], gsem)
    gfire(sidxs[0].at[1], rbs[1], gsem)

    def pair(p, _):
        for half in range(2):
            c = 2 * p + half
            for i in range(IBC):
                b = i % 4
                nb = (i + 2) % 4
                oh = half if i > 1 else 1 - half
                oi = (i - 2) % IBC

                gwait(rbs[b], gsem)

                def wait_prev():
                    pltpu.make_async_copy(
                        rbs[nb], acc.at[didxs[oh].at[oi]], ssems[nb]).wait()
                    if accc_ones is not None:
                        accc, onesv = accc_ones
                        pltpu.make_async_copy(
                            onesv, accc.at[didxs[oh].at[oi]], ssems[nb]).wait()

                if half == 0 and i < 2:
                    @pl.when(p > 0)
                    def _():
                        wait_prev()
                else:
                    wait_prev()

                if i < IBC - 2:
                    gfire(sidxs[half].at[i + 2], rbs[nb], gsem)
                elif half == 0:
                    if i == IBC - 2:
                        base = idx_base + (c + 1) * IBC
                        pltpu.sync_copy(src2.at[pl.ds(base, IBC)], sidxs[1])
                        pltpu.sync_copy(dst2.at[pl.ds(base, IBC)], didxs[1])
                    gfire(sidxs[1].at[i - (IBC - 2)], rbs[nb], gsem)
                else:
                    @pl.when(p < npair - 1)
                    def _():
                        if i == IBC - 2:
                            base = idx_base + (c + 1) * IBC
                            pltpu.sync_copy(src2.at[pl.ds(base, IBC)], sidxs[0])
                            pltpu.sync_copy(dst2.at[pl.ds(base, IBC)], didxs[0])
                        gfire(sidxs[0].at[i - (IBC - 2)], rbs[nb], gsem)

                pltpu.async_copy(rbs[b], acc.at[didxs[half].at[i]],
                                 ssems[b], add=True)
                if accc_ones is not None:
                    accc, onesv = accc_ones
                    pltpu.async_copy(onesv, accc.at[didxs[half].at[i]],
                                     ssems[b], add=True)
        return 0

    lax.fori_loop(0, npair, pair, 0)
    for i in (IBC - 2, IBC - 1):
        lb = i % 4
        pltpu.make_async_copy(rbs[lb], acc.at[didxs[1].at[i]],
                              ssems[lb]).wait()
        if accc_ones is not None:
            accc, onesv = accc_ones
            pltpu.make_async_copy(onesv, accc.at[didxs[1].at[i]],
                                  ssems[lb]).wait()


def _agg_body(t0, t1, src2, dst2, z2, sums_out,
              sidxA, sidxB, didxA, didxB, rb0, rb1, rb2, rb3,
              acc, gsem, ssem0, ssem1, ssem2, ssem3):
    cid = lax.axis_index("c")
    sid = lax.axis_index("s")

    pltpu.sync_copy(z2.at[pl.ds(sid * ZROWS, ZROWS)],
                    acc.at[pl.ds(sid * ZROWS, ZROWS)])
    plsc.subcore_barrier()

    def gfire(idx_row, rb, sem):
        @pl.when(cid == 0)
        def _():
            pltpu.async_copy(t0.at[idx_row], rb, sem)

        @pl.when(cid == 1)
        def _():
            pltpu.async_copy(t1.at[idx_row], rb, sem)

    def gwait(rb, sem):
        pltpu.make_async_copy(t0.at[sidxA.at[0]], rb, sem).wait()

    _edge_loop(gfire, gwait, RPT, src2, dst2, sid * RPT, acc, None,
               (sidxA, sidxB), (didxA, didxB), (rb0, rb1, rb2, rb3),
               gsem, (ssem0, ssem1, ssem2, ssem3))
    plsc.subcore_barrier()

    pltpu.sync_copy(acc.at[pl.ds(sid * ZROWS, ZROWS)],
                    sums_out.at[cid, pl.ds(sid * ZROWS, ZROWS)])


def _agg_counts_body(t, src2, dst2, z2, z1, sums_out, cnt_out,
                     sidxA, sidxB, didxA, didxB, rb0, rb1, rb2, rb3, onesv,
                     acc, accc, gsem, ssem0, ssem1, ssem2, ssem3):
    cid = lax.axis_index("c")
    sid = lax.axis_index("s")

    pltpu.sync_copy(z2.at[pl.ds(sid * ZROWS, ZROWS)],
                    acc.at[pl.ds(sid * ZROWS, ZROWS)])

    @pl.when(sid == 0)
    def _():
        pltpu.sync_copy(z1, accc)

    for i in range(64 // LANES):
        onesv[pl.ds(i * LANES, LANES)] = jnp.ones((LANES,), jnp.float32)

    plsc.subcore_barrier()

    def gfire(idx_row, rb, sem):
        pltpu.async_copy(t.at[idx_row], rb, sem)

    def gwait(rb, sem):
        pltpu.make_async_copy(t.at[sidxA.at[0]], rb, sem).wait()

    _edge_loop(gfire, gwait, HRPT, src2, dst2, (cid * NS + sid) * HRPT,
               acc, (accc, onesv),
               (sidxA, sidxB), (didxA, didxB), (rb0, rb1, rb2, rb3),
               gsem, (ssem0, ssem1, ssem2, ssem3))
    plsc.subcore_barrier()

    pltpu.sync_copy(acc.at[pl.ds(sid * ZROWS, ZROWS)],
                    sums_out.at[cid, pl.ds(sid * ZROWS, ZROWS)])

    @pl.when(sid == 0)
    def _():
        pltpu.sync_copy(accc, cnt_out.at[pl.ds(cid * NACC, NACC)])


def _make_agg(C, with_counts):
    mesh = plsc.VectorSubcoreMesh(core_axis_name="c", subcore_axis_name="s")
    idx = [pltpu.VMEM((IBC, 64), jnp.int32)] * 4
    rbs = [pltpu.VMEM((64, C), jnp.float32)] * 4
    sems = [pltpu.SemaphoreType.DMA] * 5
    if with_counts:
        out_type = (jax.ShapeDtypeStruct((NC, NACC, C), jnp.float32),
                    jax.ShapeDtypeStruct((NC * NACC,), jnp.float32))
        scratch = idx + rbs + [
            pltpu.VMEM((64,), jnp.float32),
            pltpu.VMEM_SHARED((NACC, C), jnp.float32),
            pltpu.VMEM_SHARED((NACC,), jnp.float32),
        ] + sems
        return pl.kernel(_agg_counts_body, out_type=out_type, mesh=mesh,
                         scratch_types=scratch)
    out_type = jax.ShapeDtypeStruct((NC, NACC, C), jnp.float32)
    scratch = idx + rbs + [
        pltpu.VMEM_SHARED((NACC, C), jnp.float32),
    ] + sems
    return pl.kernel(_agg_body, out_type=out_type, mesh=mesh,
                     scratch_types=scratch)


ROWS_TC = 1000
GRID_TC = N_NODES // ROWS_TC


def _dense1_body(ss, cc, x, wl, wr, b, oa, ob):
    inv = 1.0 / jnp.maximum(cc[0] + cc[1], 1.0)
    mean = (ss[0] + ss[1]) * inv
    h = jnp.dot(mean, wl[...], preferred_element_type=jnp.float32)
    h += jnp.dot(x[...], wr[...], preferred_element_type=jnp.float32)
    h = jnp.maximum(h + b[...], 0.0)
    oa[...] = h[:, :D_IN]
    ob[...] = h[:, D_IN:]


def _dense2_body(ss, cc, h1a, h1b, bat, wl, wr, b, wlin, blin,
                 out, pooled):
    i = pl.program_id(0)

    @pl.when(i == 0)
    def _():
        pooled[...] = jnp.zeros_like(pooled)

    inv = 1.0 / jnp.maximum(cc[0] + cc[1], 1.0)
    mean = jnp.concatenate([ss[0], ss[1]], axis=1) * inv
    h1 = jnp.concatenate([h1a[...], h1b[...]], axis=1)
    h = jnp.dot(mean, wl[...], preferred_element_type=jnp.float32)
    h += jnp.dot(h1, wr[...], preferred_element_type=jnp.float32)
    h = jnp.maximum(h + b[...], 0.0)
    oh = jnp.equal(
        bat[...],
        lax.broadcasted_iota(jnp.int32, (ROWS_TC, N_GRAPHS), 1),
    ).astype(jnp.float32)
    pooled[...] += lax.dot_general(oh, h, (((0,), (0,)), ((), ())),
                                   preferred_element_type=jnp.float32)

    @pl.when(i == GRID_TC - 1)
    def _():
        out[...] = (jnp.dot(pooled[...], wlin[...],
                            preferred_element_type=jnp.float32) + blin[...])


def _row_spec(cols):
    return pl.BlockSpec((ROWS_TC, cols), lambda i: (i, 0))


def _pair_spec(cols):
    return pl.BlockSpec((NC, ROWS_TC, cols), lambda i: (0, i, 0))


def _full_spec(r, c):
    return pl.BlockSpec((r, c), lambda i: (0, 0))


_dense1 = pl.pallas_call(
    _dense1_body,
    grid=(GRID_TC,),
    in_specs=[
        _pair_spec(D_IN), _pair_spec(1), _row_spec(D_IN),
        _full_spec(D_IN, D_HID), _full_spec(D_IN, D_HID), _full_spec(1, D_HID),
    ],
    out_specs=[_row_spec(D_IN), _row_spec(D_IN)],
    out_shape=[jax.ShapeDtypeStruct((N_NODES, D_IN), jnp.float32),
               jax.ShapeDtypeStruct((N_NODES, D_IN), jnp.float32)],
)

_dense2 = pl.pallas_call(
    _dense2_body,
    grid=(GRID_TC,),
    in_specs=[
        _pair_spec(D_IN), _pair_spec(1),
        _row_spec(D_IN), _row_spec(D_IN), _row_spec(1),
        _full_spec(D_HID, D_HID), _full_spec(D_HID, D_HID),
        _full_spec(1, D_HID), _full_spec(D_HID, 128), _full_spec(1, 128),
    ],
    out_specs=_full_spec(N_GRAPHS, 128),
    out_shape=jax.ShapeDtypeStruct((N_GRAPHS, 128), jnp.float32),
    scratch_shapes=[pltpu.VMEM((N_GRAPHS, D_HID), jnp.float32)],
)

_agg1 = _make_agg(128, with_counts=True)
_agg128 = _make_agg(128, with_counts=False)


@jax.jit
def kernel(x, edge_index, batch, W1l, b1, W1r, W2l, b2, W2r, Wlin, blin):
    src = edge_index[0]
    dst = edge_index[1]
    pad = E_PAD - N_EDGES
    pad_src = jnp.arange(pad, dtype=jnp.int32) % N_NODES
    src2 = jnp.concatenate([src, pad_src]).reshape(EROWS, 64)
    pad_dst = N_NODES + (jnp.arange(pad, dtype=jnp.int32) % (NACC - N_NODES))
    dst2 = jnp.concatenate([dst, pad_dst]).reshape(EROWS, 64)

    z128 = jnp.zeros((NACC, 128), jnp.float32)
    z1 = jnp.zeros((NACC,), jnp.float32)

    sums1, cnt = _agg1(x, src2, dst2, z128, z1)
    cnt3 = cnt.reshape(NC, NACC, 1)

    h1a, h1b = _dense1(sums1, cnt3, x, W1l, W1r, b1.reshape(1, D_HID))

    sums2 = _agg128(h1a, h1b, src2, dst2, z128)

    outp = _dense2(sums2, cnt3, h1a, h1b,
                   batch.reshape(N_NODES, 1).astype(jnp.int32),
                   W2l, W2r, b2.reshape(1, D_HID),
                   jnp.pad(Wlin, ((0, 0), (0, 128 - D_OUT))),
                   jnp.pad(blin, (0, 128 - D_OUT)).reshape(1, 128))
    return outp[:, :D_OUT]


# speedup vs baseline: 3.1887x; 1.1146x over previous
import functools

import jax
import jax.numpy as jnp
from jax import lax
from jax.experimental import pallas as pl
from jax.experimental.pallas import tpu as pltpu
from jax.experimental.pallas import tpu_sc as plsc

N_NODES = 10000
N_EDGES = 320000
D_IN = 128
D_HID = 256
D_OUT = 12
N_GRAPHS = 64

NC = 2
NS = 16
LANES = 16

EROWS = 5120
E_PAD = EROWS * 64
RPT = EROWS // NS
NACC = 10112
ZROWS = NACC // NS


IBC = 20
HRPT = EROWS // (NC * NS)


def _edge_loop(gfire, gwait, nrow, idx2, idx_base, acc, accc_ones,
               ibufs, rbs, gsem, isem, ssems):
    npair = nrow // (2 * IBC)

    pltpu.sync_copy(idx2.at[pl.ds(idx_base, IBC)], ibufs[0])
    gfire(ibufs[0].at[0, 0], rbs[0], gsem)
    gfire(ibufs[0].at[1, 0], rbs[1], gsem)

    def pair(p, _):
        for half in range(2):
            c = 2 * p + half
            nxt = 1 - half
            for i in range(IBC):
                b = i % 4
                nb = (i + 2) % 4
                oh = half if i > 1 else 1 - half
                oi = (i - 2) % IBC

                gwait(rbs[b], gsem)

                def wait_prev():
                    pltpu.make_async_copy(
                        rbs[nb], acc.at[ibufs[oh].at[oi, 1]],
                        ssems[nb]).wait()
                    if accc_ones is not None:
                        accc, onesv = accc_ones
                        pltpu.make_async_copy(
                            onesv, accc.at[ibufs[oh].at[oi, 1]],
                            ssems[nb]).wait()

                if half == 0 and i < 2:
                    @pl.when(p > 0)
                    def _():
                        wait_prev()
                else:
                    wait_prev()

                if i == 2:
                    def prefetch():
                        base = idx_base + (c + 1) * IBC
                        pltpu.async_copy(idx2.at[pl.ds(base, IBC)],
                                         ibufs[nxt], isem)
                    if half == 0:
                        prefetch()
                    else:
                        @pl.when(p < npair - 1)
                        def _():
                            prefetch()

                if i < IBC - 2:
                    gfire(ibufs[half].at[i + 2, 0], rbs[nb], gsem)
                else:
                    def next_fire():
                        if i == IBC - 2:
                            base = idx_base + (c + 1) * IBC
                            pltpu.make_async_copy(
                                idx2.at[pl.ds(base, IBC)], ibufs[nxt],
                                isem).wait()
                        gfire(ibufs[nxt].at[i - (IBC - 2), 0], rbs[nb], gsem)
                    if half == 0:
                        next_fire()
                    else:
                        @pl.when(p < npair - 1)
                        def _():
                            next_fire()

                pltpu.async_copy(rbs[b], acc.at[ibufs[half].at[i, 1]],
                                 ssems[b], add=True)
                if accc_ones is not None:
                    accc, onesv = accc_ones
                    pltpu.async_copy(onesv, accc.at[ibufs[half].at[i, 1]],
                                     ssems[b], add=True)
        return 0

    lax.fori_loop(0, npair, pair, 0)
    for i in (IBC - 2, IBC - 1):
        lb = i % 4
        pltpu.make_async_copy(rbs[lb], acc.at[ibufs[1].at[i, 1]],
                              ssems[lb]).wait()
        if accc_ones is not None:
            accc, onesv = accc_ones
            pltpu.make_async_copy(onesv, accc.at[ibufs[1].at[i, 1]],
                                  ssems[lb]).wait()


def _agg_body(t0, t1, idx2, z2, sums_out,
              ibufA, ibufB, rb0, rb1, rb2, rb3,
              acc, gsem, isem, ssem0, ssem1, ssem2, ssem3):
    cid = lax.axis_index("c")
    sid = lax.axis_index("s")

    pltpu.sync_copy(z2.at[pl.ds(sid * ZROWS, ZROWS)],
                    acc.at[pl.ds(sid * ZROWS, ZROWS)])
    plsc.subcore_barrier()

    def gfire(idx_row, rb, sem):
        @pl.when(cid == 0)
        def _():
            pltpu.async_copy(t0.at[idx_row], rb, sem)

        @pl.when(cid == 1)
        def _():
            pltpu.async_copy(t1.at[idx_row], rb, sem)

    def gwait(rb, sem):
        pltpu.make_async_copy(t0.at[ibufA.at[0, 0]], rb, sem).wait()

    _edge_loop(gfire, gwait, RPT, idx2, sid * RPT, acc, None,
               (ibufA, ibufB), (rb0, rb1, rb2, rb3),
               gsem, isem, (ssem0, ssem1, ssem2, ssem3))
    plsc.subcore_barrier()

    pltpu.sync_copy(acc.at[pl.ds(sid * ZROWS, ZROWS)],
                    sums_out.at[cid, pl.ds(sid * ZROWS, ZROWS)])


def _agg_counts_body(t, idx2, z2, z1, sums_out, cnt_out,
                     ibufA, ibufB, rb0, rb1, rb2, rb3, onesv,
                     acc, accc, gsem, isem, ssem0, ssem1, ssem2, ssem3):
    cid = lax.axis_index("c")
    sid = lax.axis_index("s")

    pltpu.sync_copy(z2.at[pl.ds(sid * ZROWS, ZROWS)],
                    acc.at[pl.ds(sid * ZROWS, ZROWS)])

    @pl.when(sid == 0)
    def _():
        pltpu.sync_copy(z1, accc)

    for i in range(64 // LANES):
        onesv[pl.ds(i * LANES, LANES)] = jnp.ones((LANES,), jnp.float32)

    plsc.subcore_barrier()

    def gfire(idx_row, rb, sem):
        pltpu.async_copy(t.at[idx_row], rb, sem)

    def gwait(rb, sem):
        pltpu.make_async_copy(t.at[ibufA.at[0, 0]], rb, sem).wait()

    _edge_loop(gfire, gwait, HRPT, idx2, (cid * NS + sid) * HRPT,
               acc, (accc, onesv),
               (ibufA, ibufB), (rb0, rb1, rb2, rb3),
               gsem, isem, (ssem0, ssem1, ssem2, ssem3))
    plsc.subcore_barrier()

    pltpu.sync_copy(acc.at[pl.ds(sid * ZROWS, ZROWS)],
                    sums_out.at[cid, pl.ds(sid * ZROWS, ZROWS)])

    @pl.when(sid == 0)
    def _():
        pltpu.sync_copy(accc, cnt_out.at[pl.ds(cid * NACC, NACC)])


def _make_agg(C, with_counts):
    mesh = plsc.VectorSubcoreMesh(core_axis_name="c", subcore_axis_name="s")
    idx = [pltpu.VMEM((IBC, 2, 64), jnp.int32)] * 2
    rbs = [pltpu.VMEM((64, C), jnp.float32)] * 4
    sems = [pltpu.SemaphoreType.DMA] * 6
    if with_counts:
        out_type = (jax.ShapeDtypeStruct((NC, NACC, C), jnp.float32),
                    jax.ShapeDtypeStruct((NC * NACC,), jnp.float32))
        scratch = idx + rbs + [
            pltpu.VMEM((64,), jnp.float32),
            pltpu.VMEM_SHARED((NACC, C), jnp.float32),
            pltpu.VMEM_SHARED((NACC,), jnp.float32),
        ] + sems
        return pl.kernel(_agg_counts_body, out_type=out_type, mesh=mesh,
                         scratch_types=scratch)
    out_type = jax.ShapeDtypeStruct((NC, NACC, C), jnp.float32)
    scratch = idx + rbs + [
        pltpu.VMEM_SHARED((NACC, C), jnp.float32),
    ] + sems
    return pl.kernel(_agg_body, out_type=out_type, mesh=mesh,
                     scratch_types=scratch)


ROWS_TC = 1000
GRID_TC = N_NODES // ROWS_TC


def _dense1_body(ss, cc, x, wl, wr, b, oa, ob):
    inv = 1.0 / jnp.maximum(cc[0] + cc[1], 1.0)
    mean = (ss[0] + ss[1]) * inv
    h = jnp.dot(mean, wl[...], preferred_element_type=jnp.float32)
    h += jnp.dot(x[...], wr[...], preferred_element_type=jnp.float32)
    h = jnp.maximum(h + b[...], 0.0)
    oa[...] = h[:, :D_IN]
    ob[...] = h[:, D_IN:]


def _dense2_body(ss, cc, h1a, h1b, bat, wl, wr, b, wlin, blin,
                 out, pooled):
    i = pl.program_id(0)

    @pl.when(i == 0)
    def _():
        pooled[...] = jnp.zeros_like(pooled)

    inv = 1.0 / jnp.maximum(cc[0] + cc[1], 1.0)
    mean = jnp.concatenate([ss[0], ss[1]], axis=1) * inv
    h1 = jnp.concatenate([h1a[...], h1b[...]], axis=1)
    h = jnp.dot(mean, wl[...], preferred_element_type=jnp.float32)
    h += jnp.dot(h1, wr[...], preferred_element_type=jnp.float32)
    h = jnp.maximum(h + b[...], 0.0)
    oh = jnp.equal(
        bat[...],
        lax.broadcasted_iota(jnp.int32, (ROWS_TC, N_GRAPHS), 1),
    ).astype(jnp.float32)
    pooled[...] += lax.dot_general(oh, h, (((0,), (0,)), ((), ())),
                                   preferred_element_type=jnp.float32)

    @pl.when(i == GRID_TC - 1)
    def _():
        out[...] = (jnp.dot(pooled[...], wlin[...],
                            preferred_element_type=jnp.float32) + blin[...])


def _row_spec(cols):
    return pl.BlockSpec((ROWS_TC, cols), lambda i: (i, 0))


def _pair_spec(cols):
    return pl.BlockSpec((NC, ROWS_TC, cols), lambda i: (0, i, 0))


def _full_spec(r, c):
    return pl.BlockSpec((r, c), lambda i: (0, 0))


_dense1 = pl.pallas_call(
    _dense1_body,
    grid=(GRID_TC,),
    in_specs=[
        _pair_spec(D_IN), _pair_spec(1), _row_spec(D_IN),
        _full_spec(D_IN, D_HID), _full_spec(D_IN, D_HID), _full_spec(1, D_HID),
    ],
    out_specs=[_row_spec(D_IN), _row_spec(D_IN)],
    out_shape=[jax.ShapeDtypeStruct((N_NODES, D_IN), jnp.float32),
               jax.ShapeDtypeStruct((N_NODES, D_IN), jnp.float32)],
)

_dense2 = pl.pallas_call(
    _dense2_body,
    grid=(GRID_TC,),
    in_specs=[
        _pair_spec(D_IN), _pair_spec(1),
        _row_spec(D_IN), _row_spec(D_IN), _row_spec(1),
        _full_spec(D_HID, D_HID), _full_spec(D_HID, D_HID),
        _full_spec(1, D_HID), _full_spec(D_HID, 128), _full_spec(1, 128),
    ],
    out_specs=_full_spec(N_GRAPHS, 128),
    out_shape=jax.ShapeDtypeStruct((N_GRAPHS, 128), jnp.float32),
    scratch_shapes=[pltpu.VMEM((N_GRAPHS, D_HID), jnp.float32)],
)

_agg1 = _make_agg(128, with_counts=True)
_agg128 = _make_agg(128, with_counts=False)


@jax.jit
def kernel(x, edge_index, batch, W1l, b1, W1r, W2l, b2, W2r, Wlin, blin):
    src = edge_index[0]
    dst = edge_index[1]
    pad = E_PAD - N_EDGES
    pad_src = jnp.arange(pad, dtype=jnp.int32) % N_NODES
    src2 = jnp.concatenate([src, pad_src]).reshape(EROWS, 64)
    pad_dst = N_NODES + (jnp.arange(pad, dtype=jnp.int32) % (NACC - N_NODES))
    dst2 = jnp.concatenate([dst, pad_dst]).reshape(EROWS, 64)
    idx2 = jnp.stack([src2, dst2], axis=1)

    z128 = jnp.zeros((NACC, 128), jnp.float32)
    z1 = jnp.zeros((NACC,), jnp.float32)

    sums1, cnt = _agg1(x, idx2, z128, z1)
    cnt3 = cnt.reshape(NC, NACC, 1)

    h1a, h1b = _dense1(sums1, cnt3, x, W1l, W1r, b1.reshape(1, D_HID))

    sums2 = _agg128(h1a, h1b, idx2, z128)

    outp = _dense2(sums2, cnt3, h1a, h1b,
                   batch.reshape(N_NODES, 1).astype(jnp.int32),
                   W2l, W2r, b2.reshape(1, D_HID),
                   jnp.pad(Wlin, ((0, 0), (0, 128 - D_OUT))),
                   jnp.pad(blin, (0, 128 - D_OUT)).reshape(1, 128))
    return outp[:, :D_OUT]


# speedup vs baseline: 3.2157x; 1.0084x over previous
import functools

import jax
import jax.numpy as jnp
from jax import lax
from jax.experimental import pallas as pl
from jax.experimental.pallas import tpu as pltpu
from jax.experimental.pallas import tpu_sc as plsc

N_NODES = 10000
N_EDGES = 320000
D_IN = 128
D_HID = 256
D_OUT = 12
N_GRAPHS = 64

NC = 2
NS = 16
LANES = 16

EROWS = 5120
E_PAD = EROWS * 64
RPT = EROWS // NS
NACC = 10112
ZROWS = NACC // NS


IBC = 10
HRPT = EROWS // (NC * NS)


def _edge_loop(gfire, gwait, nrow, idx2, idx_base, acc, accc_ones,
               ibufs, rbs, gsem, isem, ssems):
    npair = nrow // (2 * IBC)

    pltpu.sync_copy(idx2.at[pl.ds(idx_base, IBC)], ibufs[0])
    gfire(ibufs[0].at[0, 0], rbs[0], gsem)
    gfire(ibufs[0].at[1, 0], rbs[1], gsem)

    def pair(p, _):
        for half in range(2):
            c = 2 * p + half
            nxt = 1 - half
            for i in range(IBC):
                b = i % 5
                nb = (i + 2) % 5
                oh = half if i > 2 else 1 - half
                oi = (i - 3) % IBC

                gwait(rbs[b], gsem)

                def wait_prev():
                    pltpu.make_async_copy(
                        rbs[nb], acc.at[ibufs[oh].at[oi, 1]],
                        ssems[nb]).wait()
                    if accc_ones is not None:
                        accc, onesv = accc_ones
                        pltpu.make_async_copy(
                            onesv, accc.at[ibufs[oh].at[oi, 1]],
                            ssems[nb]).wait()

                if half == 0 and i < 3:
                    @pl.when(p > 0)
                    def _():
                        wait_prev()
                else:
                    wait_prev()

                if i == 2:
                    def prefetch():
                        base = idx_base + (c + 1) * IBC
                        pltpu.async_copy(idx2.at[pl.ds(base, IBC)],
                                         ibufs[nxt], isem)
                    if half == 0:
                        prefetch()
                    else:
                        @pl.when(p < npair - 1)
                        def _():
                            prefetch()

                if i < IBC - 2:
                    gfire(ibufs[half].at[i + 2, 0], rbs[nb], gsem)
                else:
                    def next_fire():
                        if i == IBC - 2:
                            base = idx_base + (c + 1) * IBC
                            pltpu.make_async_copy(
                                idx2.at[pl.ds(base, IBC)], ibufs[nxt],
                                isem).wait()
                        gfire(ibufs[nxt].at[i - (IBC - 2), 0], rbs[nb], gsem)
                    if half == 0:
                        next_fire()
                    else:
                        @pl.when(p < npair - 1)
                        def _():
                            next_fire()

                pltpu.async_copy(rbs[b], acc.at[ibufs[half].at[i, 1]],
                                 ssems[b], add=True)
                if accc_ones is not None:
                    accc, onesv = accc_ones
                    pltpu.async_copy(onesv, accc.at[ibufs[half].at[i, 1]],
                                     ssems[b], add=True)
        return 0

    lax.fori_loop(0, npair, pair, 0)
    for i in (IBC - 3, IBC - 2, IBC - 1):
        lb = i % 5
        pltpu.make_async_copy(rbs[lb], acc.at[ibufs[1].at[i, 1]],
                              ssems[lb]).wait()
        if accc_ones is not None:
            accc, onesv = accc_ones
            pltpu.make_async_copy(onesv, accc.at[ibufs[1].at[i, 1]],
                                  ssems[lb]).wait()


def _agg_body(t0, t1, idx2, z2, sums_out,
              ibufA, ibufB, rb0, rb1, rb2, rb3, rb4,
              acc, gsem, isem, ssem0, ssem1, ssem2, ssem3, ssem4):
    cid = lax.axis_index("c")
    sid = lax.axis_index("s")

    pltpu.sync_copy(z2.at[pl.ds(sid * ZROWS, ZROWS)],
                    acc.at[pl.ds(sid * ZROWS, ZROWS)])
    plsc.subcore_barrier()

    def gfire(idx_row, rb, sem):
        @pl.when(cid == 0)
        def _():
            pltpu.async_copy(t0.at[idx_row], rb, sem)

        @pl.when(cid == 1)
        def _():
            pltpu.async_copy(t1.at[idx_row], rb, sem)

    def gwait(rb, sem):
        pltpu.make_async_copy(t0.at[ibufA.at[0, 0]], rb, sem).wait()

    _edge_loop(gfire, gwait, RPT, idx2, sid * RPT, acc, None,
               (ibufA, ibufB), (rb0, rb1, rb2, rb3, rb4),
               gsem, isem, (ssem0, ssem1, ssem2, ssem3, ssem4))
    plsc.subcore_barrier()

    pltpu.sync_copy(acc.at[pl.ds(sid * ZROWS, ZROWS)],
                    sums_out.at[cid, pl.ds(sid * ZROWS, ZROWS)])


def _agg_counts_body(t, idx2, z2, z1, sums_out, cnt_out,
                     ibufA, ibufB, rb0, rb1, rb2, rb3, rb4, onesv,
                     acc, accc, gsem, isem, ssem0, ssem1, ssem2, ssem3, ssem4):
    cid = lax.axis_index("c")
    sid = lax.axis_index("s")

    pltpu.sync_copy(z2.at[pl.ds(sid * ZROWS, ZROWS)],
                    acc.at[pl.ds(sid * ZROWS, ZROWS)])

    @pl.when(sid == 0)
    def _():
        pltpu.sync_copy(z1, accc)

    for i in range(64 // LANES):
        onesv[pl.ds(i * LANES, LANES)] = jnp.ones((LANES,), jnp.float32)

    plsc.subcore_barrier()

    def gfire(idx_row, rb, sem):
        pltpu.async_copy(t.at[idx_row], rb, sem)

    def gwait(rb, sem):
        pltpu.make_async_copy(t.at[ibufA.at[0, 0]], rb, sem).wait()

    _edge_loop(gfire, gwait, HRPT, idx2, (cid * NS + sid) * HRPT,
               acc, (accc, onesv),
               (ibufA, ibufB), (rb0, rb1, rb2, rb3, rb4),
               gsem, isem, (ssem0, ssem1, ssem2, ssem3, ssem4))
    plsc.subcore_barrier()

    pltpu.sync_copy(acc.at[pl.ds(sid * ZROWS, ZROWS)],
                    sums_out.at[cid, pl.ds(sid * ZROWS, ZROWS)])

    @pl.when(sid == 0)
    def _():
        pltpu.sync_copy(accc, cnt_out.at[pl.ds(cid * NACC, NACC)])


def _make_agg(C, with_counts):
    mesh = plsc.VectorSubcoreMesh(core_axis_name="c", subcore_axis_name="s")
    idx = [pltpu.VMEM((IBC, 2, 64), jnp.int32)] * 2
    rbs = [pltpu.VMEM((64, C), jnp.float32)] * 5
    sems = [pltpu.SemaphoreType.DMA] * 7
    if with_counts:
        out_type = (jax.ShapeDtypeStruct((NC, NACC, C), jnp.float32),
                    jax.ShapeDtypeStruct((NC * NACC,), jnp.float32))
        scratch = idx + rbs + [
            pltpu.VMEM((64,), jnp.float32),
            pltpu.VMEM_SHARED((NACC, C), jnp.float32),
            pltpu.VMEM_SHARED((NACC,), jnp.float32),
        ] + sems
        return pl.kernel(_agg_counts_body, out_type=out_type, mesh=mesh,
                         scratch_types=scratch)
    out_type = jax.ShapeDtypeStruct((NC, NACC, C), jnp.float32)
    scratch = idx + rbs + [
        pltpu.VMEM_SHARED((NACC, C), jnp.float32),
    ] + sems
    return pl.kernel(_agg_body, out_type=out_type, mesh=mesh,
                     scratch_types=scratch)


ROWS_TC = 1000
GRID_TC = N_NODES // ROWS_TC


def _dense1_body(ss, cc, x, wl, wr, b, oa, ob):
    inv = 1.0 / jnp.maximum(cc[0] + cc[1], 1.0)
    mean = (ss[0] + ss[1]) * inv
    h = jnp.dot(mean, wl[...], preferred_element_type=jnp.float32)
    h += jnp.dot(x[...], wr[...], preferred_element_type=jnp.float32)
    h = jnp.maximum(h + b[...], 0.0)
    oa[...] = h[:, :D_IN]
    ob[...] = h[:, D_IN:]


def _dense2_body(ss, cc, h1a, h1b, bat, wl, wr, b, wlin, blin,
                 out, pooled):
    i = pl.program_id(0)

    @pl.when(i == 0)
    def _():
        pooled[...] = jnp.zeros_like(pooled)

    inv = 1.0 / jnp.maximum(cc[0] + cc[1], 1.0)
    mean = jnp.concatenate([ss[0], ss[1]], axis=1) * inv
    h1 = jnp.concatenate([h1a[...], h1b[...]], axis=1)
    h = jnp.dot(mean, wl[...], preferred_element_type=jnp.float32)
    h += jnp.dot(h1, wr[...], preferred_element_type=jnp.float32)
    h = jnp.maximum(h + b[...], 0.0)
    oh = jnp.equal(
        bat[...],
        lax.broadcasted_iota(jnp.int32, (ROWS_TC, N_GRAPHS), 1),
    ).astype(jnp.float32)
    pooled[...] += lax.dot_general(oh, h, (((0,), (0,)), ((), ())),
                                   preferred_element_type=jnp.float32)

    @pl.when(i == GRID_TC - 1)
    def _():
        out[...] = (jnp.dot(pooled[...], wlin[...],
                            preferred_element_type=jnp.float32) + blin[...])


def _row_spec(cols):
    return pl.BlockSpec((ROWS_TC, cols), lambda i: (i, 0))


def _pair_spec(cols):
    return pl.BlockSpec((NC, ROWS_TC, cols), lambda i: (0, i, 0))


def _full_spec(r, c):
    return pl.BlockSpec((r, c), lambda i: (0, 0))


_dense1 = pl.pallas_call(
    _dense1_body,
    grid=(GRID_TC,),
    in_specs=[
        _pair_spec(D_IN), _pair_spec(1), _row_spec(D_IN),
        _full_spec(D_IN, D_HID), _full_spec(D_IN, D_HID), _full_spec(1, D_HID),
    ],
    out_specs=[_row_spec(D_IN), _row_spec(D_IN)],
    out_shape=[jax.ShapeDtypeStruct((N_NODES, D_IN), jnp.float32),
               jax.ShapeDtypeStruct((N_NODES, D_IN), jnp.float32)],
)

_dense2 = pl.pallas_call(
    _dense2_body,
    grid=(GRID_TC,),
    in_specs=[
        _pair_spec(D_IN), _pair_spec(1),
        _row_spec(D_IN), _row_spec(D_IN), _row_spec(1),
        _full_spec(D_HID, D_HID), _full_spec(D_HID, D_HID),
        _full_spec(1, D_HID), _full_spec(D_HID, 128), _full_spec(1, 128),
    ],
    out_specs=_full_spec(N_GRAPHS, 128),
    out_shape=jax.ShapeDtypeStruct((N_GRAPHS, 128), jnp.float32),
    scratch_shapes=[pltpu.VMEM((N_GRAPHS, D_HID), jnp.float32)],
)

_agg1 = _make_agg(128, with_counts=True)
_agg128 = _make_agg(128, with_counts=False)


@jax.jit
def kernel(x, edge_index, batch, W1l, b1, W1r, W2l, b2, W2r, Wlin, blin):
    src = edge_index[0]
    dst = edge_index[1]
    pad = E_PAD - N_EDGES
    pad_src = jnp.arange(pad, dtype=jnp.int32) % N_NODES
    src2 = jnp.concatenate([src, pad_src]).reshape(EROWS, 64)
    pad_dst = N_NODES + (jnp.arange(pad, dtype=jnp.int32) % (NACC - N_NODES))
    dst2 = jnp.concatenate([dst, pad_dst]).reshape(EROWS, 64)
    idx2 = jnp.stack([src2, dst2], axis=1)

    z128 = jnp.zeros((NACC, 128), jnp.float32)
    z1 = jnp.zeros((NACC,), jnp.float32)

    sums1, cnt = _agg1(x, idx2, z128, z1)
    cnt3 = cnt.reshape(NC, NACC, 1)

    h1a, h1b = _dense1(sums1, cnt3, x, W1l, W1r, b1.reshape(1, D_HID))

    sums2 = _agg128(h1a, h1b, idx2, z128)

    outp = _dense2(sums2, cnt3, h1a, h1b,
                   batch.reshape(N_NODES, 1).astype(jnp.int32),
                   W2l, W2r, b2.reshape(1, D_HID),
                   jnp.pad(Wlin, ((0, 0), (0, 128 - D_OUT))),
                   jnp.pad(blin, (0, 128 - D_OUT)).reshape(1, 128))
    return outp[:, :D_OUT]
